# baseline (speedup 1.0000x reference)
import functools
import jax
import jax.numpy as jnp
from jax import lax
from jax.experimental import pallas as pl
from jax.experimental.pallas import tpu as pltpu
from jax.experimental.pallas import tpu_sc as plsc

B, L, D = 4, 2048, 768
NW = 32
L_SC = 256
L0_SC = L - L_SC
CH = L_SC // 8
VL = 16
BL = 896


def _sc_part(x, emb):
    mesh = plsc.VectorSubcoreMesh(core_axis_name="c", subcore_axis_name="s")

    @functools.partial(
        pl.kernel,
        mesh=mesh,
        out_type=jax.ShapeDtypeStruct((B, L_SC, D), jnp.float32),
        scratch_types=[
            pltpu.VMEM((CH, D), jnp.float32),
            pltpu.VMEM((CH, D), jnp.float32),
            pltpu.SemaphoreType.DMA,
            pltpu.SemaphoreType.DMA,
        ],
    )
    def k(x_hbm, emb_hbm, out_hbm, bufe, bufx, sem_e, sem_x):
        wid = lax.axis_index("s") * 2 + lax.axis_index("c")
        b = wid // 8
        l0 = lax.rem(wid, 8) * CH
        he = pltpu.async_copy(emb_hbm.at[pl.ds(L0_SC + l0, CH)], bufe, sem_e)
        hx = pltpu.async_copy(x_hbm.at[b, pl.ds(L0_SC + l0, CH)], bufx, sem_x)
        he.wait()
        hx.wait()

        @plsc.parallel_loop(0, CH, step=1, unroll=2)
        def _add(r):
            for cc in range(0, D, VL):
                plsc.addupdate(bufx.at[r, pl.ds(cc, VL)], bufe[r, pl.ds(cc, VL)])

        pltpu.sync_copy(bufx, out_hbm.at[b, pl.ds(l0, CH)])

    return k(x, emb)


def _tc_body(x_ref, e_ref, o_ref):
    o_ref[...] = x_ref[...] + e_ref[...][None, :, :]


def _tc_part(x, emb):
    return pl.pallas_call(
        _tc_body,
        grid=((L - L_SC) // BL,),
        in_specs=[
            pl.BlockSpec((B, BL, D), lambda i: (0, i, 0)),
            pl.BlockSpec((BL, D), lambda i: (i, 0)),
        ],
        out_specs=pl.BlockSpec((B, BL, D), lambda i: (0, i, 0)),
        out_shape=jax.ShapeDtypeStruct((B, L, D), x.dtype),
    )(x, emb)


def kernel(x, row_embed):
    out_sc = _sc_part(x, row_embed)
    out_tc = _tc_part(x, row_embed)
    return lax.dynamic_update_slice(out_tc, out_sc, (0, L0_SC, 0))


# speedup vs baseline: 1.0058x; 1.0058x over previous
import functools
import jax
import jax.numpy as jnp
from jax import lax
from jax.experimental import pallas as pl
from jax.experimental.pallas import tpu as pltpu
from jax.experimental.pallas import tpu_sc as plsc

B, L, D = 4, 2048, 768
NW = 32
L_SC = 256
L0_SC = L - L_SC
CH = L_SC // 8
VL = 16
BL = 448


def _sc_part(x, emb):
    mesh = plsc.VectorSubcoreMesh(core_axis_name="c", subcore_axis_name="s")

    @functools.partial(
        pl.kernel,
        mesh=mesh,
        out_type=jax.ShapeDtypeStruct((B, L_SC, D), jnp.float32),
        scratch_types=[
            pltpu.VMEM((CH, D), jnp.float32),
            pltpu.VMEM((CH, D), jnp.float32),
            pltpu.SemaphoreType.DMA,
            pltpu.SemaphoreType.DMA,
        ],
    )
    def k(x_hbm, emb_hbm, out_hbm, bufe, bufx, sem_e, sem_x):
        wid = lax.axis_index("s") * 2 + lax.axis_index("c")
        b = wid // 8
        l0 = lax.rem(wid, 8) * CH
        he = pltpu.async_copy(emb_hbm.at[pl.ds(L0_SC + l0, CH)], bufe, sem_e)
        hx = pltpu.async_copy(x_hbm.at[b, pl.ds(L0_SC + l0, CH)], bufx, sem_x)
        he.wait()
        hx.wait()

        @plsc.parallel_loop(0, CH, step=1, unroll=2)
        def _add(r):
            for cc in range(0, D, VL):
                plsc.addupdate(bufx.at[r, pl.ds(cc, VL)], bufe[r, pl.ds(cc, VL)])

        pltpu.sync_copy(bufx, out_hbm.at[b, pl.ds(l0, CH)])

    return k(x, emb)


def _tc_body(x_ref, e_ref, o_ref):
    o_ref[...] = x_ref[...] + e_ref[...][None, :, :]


def _tc_part(x, emb):
    return pl.pallas_call(
        _tc_body,
        grid=((L - L_SC) // BL,),
        in_specs=[
            pl.BlockSpec((B, BL, D), lambda i: (0, i, 0)),
            pl.BlockSpec((BL, D), lambda i: (i, 0)),
        ],
        out_specs=pl.BlockSpec((B, BL, D), lambda i: (0, i, 0)),
        out_shape=jax.ShapeDtypeStruct((B, L, D), x.dtype),
    )(x, emb)


def kernel(x, row_embed):
    out_sc = _sc_part(x, row_embed)
    out_tc = _tc_part(x, row_embed)
    return lax.dynamic_update_slice(out_tc, out_sc, (0, L0_SC, 0))
